# Initial kernel scaffold; baseline (speedup 1.0000x reference)
import jax
import jax.numpy as jnp
from jax.experimental import pallas as pl


def kernel(X, edges, W_init, b_init, W_cls, b_cls):
    raise NotImplementedError("write your pallas kernel here")


# speedup vs baseline: 25.8409x; 25.8409x over previous
import functools

import jax
import jax.numpy as jnp
from jax import lax
from jax.experimental import pallas as pl
from jax.experimental.pallas import tpu as pltpu
from jax.experimental.pallas import tpu_sc as plsc

N = 10000
E = 320000
D = 128
K = 4
F = 32
ROUTIT = 6
LAYERS = 4
NCLS = 40

NC = 2
NS = 16
NW = NC * NS
L = 16

NPAD = 10112
NRW = 320
NRW_LAST = NPAD - (NW - 1) * NRW
RPT = NPAD // NS
C = 128
NCHUNK = 79
EPW = NCHUNK * C
EPADT = EPW * NW

_MESH = plsc.VectorSubcoreMesh(
    core_axis_name="c", subcore_axis_name="s", num_cores=NC, num_subcores=NS
)
_SC_PARAMS = pltpu.CompilerParams(needs_layout_passes=False)


def _rsqrt(x):
    i = plsc.bitcast(x, jnp.int32)
    y = plsc.bitcast(jnp.int32(0x5F3759DF) - (i >> 1), jnp.float32)
    for _ in range(3):
        y = y * (1.5 - 0.5 * x * y * y)
    return y


def _edge_body(z_hbm, un_hbm, srcI, dstI, zeros_hbm, agg_out,
               srcb, dstb, zr, ur, wb, pbuf, agg_sp, sem):
    c = lax.axis_index("c")
    s = lax.axis_index("s")
    w = c * NS + s
    pltpu.sync_copy(zeros_hbm, agg_sp.at[pl.ds(s * RPT, RPT)])
    plsc.subcore_barrier()

    mask15 = lax.iota(jnp.int32, 16) == 15

    def chunk(j, _):
        base = w * NCHUNK + j
        pltpu.sync_copy(srcI.at[base], srcb)
        pltpu.sync_copy(dstI.at[base], dstb)
        pltpu.async_copy(z_hbm.at[srcb.at[0]], zr, sem).wait()
        pltpu.async_copy(un_hbm.at[dstb.at[0]], ur, sem).wait()

        def pass_a(e, carry):
            for k in range(K):
                m0 = zr[e, pl.ds(2 * k * L, L)] * ur[e, pl.ds(2 * k * L, L)]
                m1 = (zr[e, pl.ds((2 * k + 1) * L, L)]
                      * ur[e, pl.ds((2 * k + 1) * L, L)])
                t = plsc.cumsum(m0 + m1)
                idx = jnp.broadcast_to(k * C + e, (L,)).astype(jnp.int32)
                plsc.store_scatter(pbuf, [idx], t, mask=mask15)
            return carry

        lax.fori_loop(0, C, pass_a, 0)

        for g in range(C // L):
            ps = [jnp.exp(pbuf[pl.ds(k * C + g * L, L)]) for k in range(K)]
            tot = (ps[0] + ps[1]) + (ps[2] + ps[3])
            inv = 1.0 / tot
            for k in range(K):
                pbuf[pl.ds(k * C + g * L, L)] = ps[k] * inv

        def pass_c(e, carry):
            ev = jnp.broadcast_to(e, (L,)).astype(jnp.int32)
            for k in range(K):
                pv = plsc.load_gather(pbuf, [ev + k * C])
                wb[e, pl.ds(2 * k * L, L)] = zr[e, pl.ds(2 * k * L, L)] * pv
                wb[e, pl.ds((2 * k + 1) * L, L)] = (
                    zr[e, pl.ds((2 * k + 1) * L, L)] * pv)
            return carry

        lax.fori_loop(0, C, pass_c, 0)

        pltpu.sync_copy(wb, agg_sp.at[dstb.at[0]], add=True)
        return _

    lax.fori_loop(0, NCHUNK, chunk, 0)
    plsc.subcore_barrier()
    pltpu.sync_copy(agg_sp.at[pl.ds(s * RPT, RPT)],
                    agg_out.at[c, pl.ds(s * RPT, RPT)])


_edge_kernel = functools.partial(
    pl.kernel,
    out_type=jax.ShapeDtypeStruct((NC, NPAD, D), jnp.float32),
    mesh=_MESH,
    compiler_params=_SC_PARAMS,
    scratch_types=[
        pltpu.VMEM((1, C), jnp.int32),
        pltpu.VMEM((1, C), jnp.int32),
        pltpu.VMEM((C, D), jnp.float32),
        pltpu.VMEM((C, D), jnp.float32),
        pltpu.VMEM((C, D), jnp.float32),
        pltpu.VMEM((K * C,), jnp.float32),
        pltpu.VMEM_SHARED((NPAD, D), jnp.float32),
        pltpu.SemaphoreType.DMA,
    ],
)(_edge_body)


def _node_body(final, zraw_hbm, agg_hbm, out1, out2, zb, aa, ab):
    c = lax.axis_index("c")
    s = lax.axis_index("s")
    w = c * NS + s

    @pl.when(w < NW - 1)
    def _():
        _node_rows(final, NRW, w * NRW, zraw_hbm, agg_hbm, out1, out2,
                   zb, aa, ab)

    @pl.when(w == NW - 1)
    def _():
        _node_rows(final, NRW_LAST, (NW - 1) * NRW, zraw_hbm, agg_hbm,
                   out1, out2, zb, aa, ab)


def _node_rows(final, nrows, r0, zraw_hbm, agg_hbm, out1, out2, zb, aa, ab):
    pltpu.sync_copy(zraw_hbm.at[pl.ds(r0, nrows)], zb.at[pl.ds(0, nrows)])
    pltpu.sync_copy(agg_hbm.at[0, pl.ds(r0, nrows)], aa.at[pl.ds(0, nrows)])
    pltpu.sync_copy(agg_hbm.at[1, pl.ds(r0, nrows)], ab.at[pl.ds(0, nrows)])

    def row(r, carry):
        for k in range(K):
            a = (zb[r, pl.ds(2 * k * L, L)] + aa[r, pl.ds(2 * k * L, L)]
                 + ab[r, pl.ds(2 * k * L, L)])
            b = (zb[r, pl.ds((2 * k + 1) * L, L)]
                 + aa[r, pl.ds((2 * k + 1) * L, L)]
                 + ab[r, pl.ds((2 * k + 1) * L, L)])
            ss = jnp.sum(a * a) + jnp.sum(b * b)
            ssv = jnp.broadcast_to(ss, (L,))
            sc1 = _rsqrt(jnp.maximum(ssv, 1e-24))
            ua = a * sc1
            ub = b * sc1
            if final:
                ra = jnp.maximum(ua, 0.0)
                rb = jnp.maximum(ub, 0.0)
                ssr = jnp.sum(ra * ra) + jnp.sum(rb * rb)
                sc3 = _rsqrt(jnp.maximum(jnp.broadcast_to(ssr, (L,)), 1e-24))
                zb[r, pl.ds(2 * k * L, L)] = ra
                zb[r, pl.ds((2 * k + 1) * L, L)] = rb
                aa[r, pl.ds(2 * k * L, L)] = ra * sc3
                aa[r, pl.ds((2 * k + 1) * L, L)] = rb * sc3
            else:
                ssv2 = ssv * sc1 * sc1
                sc2 = _rsqrt(jnp.maximum(ssv2, 1e-24))
                zb[r, pl.ds(2 * k * L, L)] = ua * sc2
                zb[r, pl.ds((2 * k + 1) * L, L)] = ub * sc2
        return carry

    lax.fori_loop(0, nrows, row, 0)
    pltpu.sync_copy(zb.at[pl.ds(0, nrows)], out1.at[pl.ds(r0, nrows)])
    if final:
        pltpu.sync_copy(aa.at[pl.ds(0, nrows)], out2.at[pl.ds(r0, nrows)])


def _make_node_kernel(final):
    nout = 2 if final else 1
    out_type = [jax.ShapeDtypeStruct((NPAD, D), jnp.float32)] * nout
    if not final:
        out_type = out_type[0]

    def body(zraw_hbm, agg_hbm, *rest):
        if final:
            out1, out2, zb, aa, ab = rest
        else:
            (out1, zb, aa, ab) = rest
            out2 = None
        _node_body(final, zraw_hbm, agg_hbm, out1, out2, zb, aa, ab)

    return functools.partial(
        pl.kernel,
        out_type=out_type,
        mesh=_MESH,
        compiler_params=_SC_PARAMS,
        scratch_types=[
            pltpu.VMEM((NRW, D), jnp.float32),
            pltpu.VMEM((NRW, D), jnp.float32),
            pltpu.VMEM((NRW, D), jnp.float32),
        ],
    )(body)


_node_mid = _make_node_kernel(False)
_node_final = _make_node_kernel(True)


def _init_tc(x_ref, w_ref, b_ref, m_ref, mt_ref, zraw_ref, z0_ref):
    x = x_ref[...]
    h = jnp.maximum(
        jnp.dot(x, w_ref[...], preferred_element_type=jnp.float32)
        + b_ref[...], 0.0)
    ss = jnp.dot(h * h, m_ref[...], preferred_element_type=jnp.float32)
    sc = 1.0 / jnp.maximum(jnp.sqrt(ss), 1e-12)
    zraw = h * jnp.dot(sc, mt_ref[...], preferred_element_type=jnp.float32)
    zraw_ref[...] = zraw
    ss2 = jnp.dot(zraw * zraw, m_ref[...], preferred_element_type=jnp.float32)
    sc2 = 1.0 / jnp.maximum(jnp.sqrt(ss2), 1e-12)
    z0_ref[...] = zraw * jnp.dot(
        sc2, mt_ref[...], preferred_element_type=jnp.float32)


def _cls_tc(zf_ref, w_ref, b_ref, o_ref):
    o_ref[...] = (
        jnp.dot(zf_ref[...], w_ref[...], preferred_element_type=jnp.float32)
        + b_ref[...])


def kernel(X, edges, W_init, b_init, W_cls, b_cls):
    f32 = jnp.float32
    Xp = jnp.pad(X, ((0, NPAD - N), (0, 0)))
    W0 = W_init.transpose(1, 0, 2).reshape(D, D)
    b0 = b_init.reshape(1, D)
    Mfac = jnp.kron(jnp.eye(K, dtype=f32), jnp.ones((F, 1), f32))
    MfacT = Mfac.T

    src = edges[0].astype(jnp.int32)
    dst = edges[1].astype(jnp.int32)
    padv = (N + (jnp.arange(EPADT - E, dtype=jnp.int32) % (NPAD - N)))
    srcp = jnp.concatenate([src, padv]).reshape(NW * NCHUNK, 1, C)
    dstp = jnp.concatenate([dst, padv]).reshape(NW * NCHUNK, 1, C)
    zeros_tile = jnp.zeros((RPT, D), f32)

    Zraw, z = pl.pallas_call(
        _init_tc,
        out_shape=[jax.ShapeDtypeStruct((NPAD, D), f32)] * 2,
    )(Xp, W0, b0, Mfac, MfacT)

    for _layer in range(LAYERS):
        un = z
        for t in range(ROUTIT):
            agg = _edge_kernel(z, un, srcp, dstp, zeros_tile)
            if t < ROUTIT - 1:
                un = _node_mid(Zraw, agg)
            else:
                Zraw, z = _node_final(Zraw, agg)

    Wp = jnp.pad(W_cls, ((0, 0), (0, D - NCLS)))
    bp = jnp.pad(b_cls, (0, D - NCLS)).reshape(1, D)
    out = pl.pallas_call(
        _cls_tc,
        out_shape=jax.ShapeDtypeStruct((NPAD, D), f32),
    )(Zraw, Wp, bp)
    return out[:N, :NCLS]


# speedup vs baseline: 66.9615x; 2.5913x over previous
import functools

import jax
import jax.numpy as jnp
from jax import lax
from jax.experimental import pallas as pl
from jax.experimental.pallas import tpu as pltpu
from jax.experimental.pallas import tpu_sc as plsc

N = 10000
E = 320000
D = 128
K = 4
F = 32
ROUTIT = 6
LAYERS = 4
NCLS = 40

NC = 2
NS = 16
NW = NC * NS
L = 16

NPAD = 10112
NRW = 320
NRW_LAST = NPAD - (NW - 1) * NRW
RPT = NPAD // NS
C = 128
NCHUNK = 80
EPW = NCHUNK * C
EPADT = EPW * NW

_MESH = plsc.VectorSubcoreMesh(
    core_axis_name="c", subcore_axis_name="s", num_cores=NC, num_subcores=NS
)
_SC_PARAMS = pltpu.CompilerParams(needs_layout_passes=False)


def _rsqrt(x):
    i = plsc.bitcast(x, jnp.int32)
    y = plsc.bitcast(jnp.int32(0x5F3759DF) - (i >> 1), jnp.float32)
    for _ in range(3):
        y = y * (1.5 - 0.5 * x * y * y)
    return y


def _edge_body(z_hbm, un_hbm, srcI, dstI, zeros_hbm, agg_out,
               srcb, dstb, zr, ur, wb, pbuf, agg_sp, sem):
    c = lax.axis_index("c")
    s = lax.axis_index("s")
    w = c * NS + s
    pltpu.sync_copy(zeros_hbm, agg_sp.at[pl.ds(s * RPT, RPT)])
    plsc.subcore_barrier()

    mask15 = lax.iota(jnp.int32, 16) == 15

    def chunk(j, carry):
        base = w * NCHUNK + j
        pltpu.sync_copy(srcI.at[base], srcb)
        pltpu.sync_copy(dstI.at[base], dstb)
        dz = pltpu.async_copy(z_hbm.at[srcb.at[0]], zr, sem)
        du = pltpu.async_copy(un_hbm.at[dstb.at[0]], ur, sem)
        dz.wait()
        du.wait()

        @plsc.parallel_loop(0, C, unroll=2)
        def _(e):
            for k in range(K):
                m0 = zr[e, pl.ds(2 * k * L, L)] * ur[e, pl.ds(2 * k * L, L)]
                m1 = (zr[e, pl.ds((2 * k + 1) * L, L)]
                      * ur[e, pl.ds((2 * k + 1) * L, L)])
                t = plsc.cumsum(m0 + m1)
                idx = jnp.broadcast_to(k * C + e, (L,)).astype(jnp.int32)
                plsc.store_scatter(pbuf, [idx], t, mask=mask15)

        for g in range(C // L):
            ps = [jnp.exp(pbuf[pl.ds(k * C + g * L, L)]) for k in range(K)]
            tot = (ps[0] + ps[1]) + (ps[2] + ps[3])
            inv = 1.0 / tot
            for k in range(K):
                pbuf[pl.ds(k * C + g * L, L)] = ps[k] * inv

        @plsc.parallel_loop(0, C, unroll=2)
        def _(e):
            ev = jnp.broadcast_to(e, (L,)).astype(jnp.int32)
            for k in range(K):
                pv = plsc.load_gather(pbuf, [ev + k * C])
                wb[e, pl.ds(2 * k * L, L)] = zr[e, pl.ds(2 * k * L, L)] * pv
                wb[e, pl.ds((2 * k + 1) * L, L)] = (
                    zr[e, pl.ds((2 * k + 1) * L, L)] * pv)

        pltpu.sync_copy(wb, agg_sp.at[dstb.at[0]], add=True)
        return carry

    lax.fori_loop(0, NCHUNK, chunk, 0)
    plsc.subcore_barrier()
    pltpu.sync_copy(agg_sp.at[pl.ds(s * RPT, RPT)],
                    agg_out.at[c, pl.ds(s * RPT, RPT)])


_edge_kernel = functools.partial(
    pl.kernel,
    out_type=jax.ShapeDtypeStruct((NC, NPAD, D), jnp.float32),
    mesh=_MESH,
    compiler_params=_SC_PARAMS,
    scratch_types=[
        pltpu.VMEM((1, C), jnp.int32),
        pltpu.VMEM((1, C), jnp.int32),
        pltpu.VMEM((C, D), jnp.float32),
        pltpu.VMEM((C, D), jnp.float32),
        pltpu.VMEM((C, D), jnp.float32),
        pltpu.VMEM((K * C,), jnp.float32),
        pltpu.VMEM_SHARED((NPAD, D), jnp.float32),
        pltpu.SemaphoreType.DMA,
    ],
)(_edge_body)


def _node_body(final, zraw_hbm, agg_hbm, out1, out2, zb, aa, ab):
    c = lax.axis_index("c")
    s = lax.axis_index("s")
    w = c * NS + s

    @pl.when(w < NW - 1)
    def _():
        _node_rows(final, NRW, w * NRW, zraw_hbm, agg_hbm, out1, out2,
                   zb, aa, ab)

    @pl.when(w == NW - 1)
    def _():
        _node_rows(final, NRW_LAST, (NW - 1) * NRW, zraw_hbm, agg_hbm,
                   out1, out2, zb, aa, ab)


def _node_rows(final, nrows, r0, zraw_hbm, agg_hbm, out1, out2, zb, aa, ab):
    pltpu.sync_copy(zraw_hbm.at[pl.ds(r0, nrows)], zb.at[pl.ds(0, nrows)])
    pltpu.sync_copy(agg_hbm.at[0, pl.ds(r0, nrows)], aa.at[pl.ds(0, nrows)])
    pltpu.sync_copy(agg_hbm.at[1, pl.ds(r0, nrows)], ab.at[pl.ds(0, nrows)])

    def row(r, carry):
        for k in range(K):
            a = (zb[r, pl.ds(2 * k * L, L)] + aa[r, pl.ds(2 * k * L, L)]
                 + ab[r, pl.ds(2 * k * L, L)])
            b = (zb[r, pl.ds((2 * k + 1) * L, L)]
                 + aa[r, pl.ds((2 * k + 1) * L, L)]
                 + ab[r, pl.ds((2 * k + 1) * L, L)])
            ss = jnp.sum(a * a) + jnp.sum(b * b)
            ssv = jnp.broadcast_to(ss, (L,))
            sc1 = _rsqrt(jnp.maximum(ssv, 1e-24))
            ua = a * sc1
            ub = b * sc1
            if final:
                ra = jnp.maximum(ua, 0.0)
                rb = jnp.maximum(ub, 0.0)
                ssr = jnp.sum(ra * ra) + jnp.sum(rb * rb)
                sc3 = _rsqrt(jnp.maximum(jnp.broadcast_to(ssr, (L,)), 1e-24))
                zb[r, pl.ds(2 * k * L, L)] = ra
                zb[r, pl.ds((2 * k + 1) * L, L)] = rb
                aa[r, pl.ds(2 * k * L, L)] = ra * sc3
                aa[r, pl.ds((2 * k + 1) * L, L)] = rb * sc3
            else:
                ssv2 = ssv * sc1 * sc1
                sc2 = _rsqrt(jnp.maximum(ssv2, 1e-24))
                zb[r, pl.ds(2 * k * L, L)] = ua * sc2
                zb[r, pl.ds((2 * k + 1) * L, L)] = ub * sc2
        return carry

    lax.fori_loop(0, nrows, row, 0)
    pltpu.sync_copy(zb.at[pl.ds(0, nrows)], out1.at[pl.ds(r0, nrows)])
    if final:
        pltpu.sync_copy(aa.at[pl.ds(0, nrows)], out2.at[pl.ds(r0, nrows)])


def _make_node_kernel(final):
    nout = 2 if final else 1
    out_type = [jax.ShapeDtypeStruct((NPAD, D), jnp.float32)] * nout
    if not final:
        out_type = out_type[0]

    def body(zraw_hbm, agg_hbm, *rest):
        if final:
            out1, out2, zb, aa, ab = rest
        else:
            (out1, zb, aa, ab) = rest
            out2 = None
        _node_body(final, zraw_hbm, agg_hbm, out1, out2, zb, aa, ab)

    return functools.partial(
        pl.kernel,
        out_type=out_type,
        mesh=_MESH,
        compiler_params=_SC_PARAMS,
        scratch_types=[
            pltpu.VMEM((NRW, D), jnp.float32),
            pltpu.VMEM((NRW, D), jnp.float32),
            pltpu.VMEM((NRW, D), jnp.float32),
        ],
    )(body)


_node_mid = _make_node_kernel(False)
_node_final = _make_node_kernel(True)


def _init_tc(x_ref, w_ref, b_ref, m_ref, mt_ref, zraw_ref, z0_ref):
    x = x_ref[...]
    h = jnp.maximum(
        jnp.dot(x, w_ref[...], preferred_element_type=jnp.float32)
        + b_ref[...], 0.0)
    ss = jnp.dot(h * h, m_ref[...], preferred_element_type=jnp.float32)
    sc = 1.0 / jnp.maximum(jnp.sqrt(ss), 1e-12)
    zraw = h * jnp.dot(sc, mt_ref[...], preferred_element_type=jnp.float32)
    zraw_ref[...] = zraw
    ss2 = jnp.dot(zraw * zraw, m_ref[...], preferred_element_type=jnp.float32)
    sc2 = 1.0 / jnp.maximum(jnp.sqrt(ss2), 1e-12)
    z0_ref[...] = zraw * jnp.dot(
        sc2, mt_ref[...], preferred_element_type=jnp.float32)


def _cls_tc(zf_ref, w_ref, b_ref, o_ref):
    o_ref[...] = (
        jnp.dot(zf_ref[...], w_ref[...], preferred_element_type=jnp.float32)
        + b_ref[...])


def kernel(X, edges, W_init, b_init, W_cls, b_cls):
    f32 = jnp.float32
    Xp = jnp.pad(X, ((0, NPAD - N), (0, 0)))
    W0 = W_init.transpose(1, 0, 2).reshape(D, D)
    b0 = b_init.reshape(1, D)
    Mfac = jnp.kron(jnp.eye(K, dtype=f32), jnp.ones((F, 1), f32))
    MfacT = Mfac.T

    src = edges[0].astype(jnp.int32)
    dst = edges[1].astype(jnp.int32)
    padv = (N + (jnp.arange(EPADT - E, dtype=jnp.int32) % (NPAD - N)))
    srcp = jnp.concatenate([src, padv]).reshape(NW * NCHUNK, 1, C)
    dstp = jnp.concatenate([dst, padv]).reshape(NW * NCHUNK, 1, C)
    zeros_tile = jnp.zeros((RPT, D), f32)

    Zraw, z = pl.pallas_call(
        _init_tc,
        out_shape=[jax.ShapeDtypeStruct((NPAD, D), f32)] * 2,
    )(Xp, W0, b0, Mfac, MfacT)

    for _layer in range(LAYERS):
        un = z
        for t in range(ROUTIT):
            agg = _edge_kernel(z, un, srcp, dstp, zeros_tile)
            if t < ROUTIT - 1:
                un = _node_mid(Zraw, agg)
            else:
                Zraw, z = _node_final(Zraw, agg)

    Wp = jnp.pad(W_cls, ((0, 0), (0, D - NCLS)))
    bp = jnp.pad(b_cls, (0, D - NCLS)).reshape(1, D)
    out = pl.pallas_call(
        _cls_tc,
        out_shape=jax.ShapeDtypeStruct((NPAD, D), f32),
    )(Zraw, Wp, bp)
    return out[:N, :NCLS]
